# baseline (speedup 1.0000x reference)
import jax
import jax.numpy as jnp
from jax.experimental import pallas as pl

_B, _D, _H, _W = 2, 256, 56, 56
_K = 1024
_N = _B * _H * _W
_RB = 896
_H2 = 2 * _H
_N2 = _B * _H2 * _H2
_RH = 16


def _vq_body(z_ref, e_ref, o_ref):
    zb = z_ref[...]
    e = e_ref[...]
    zsq = jnp.sum(zb * zb, axis=1, keepdims=True)
    esq = jnp.sum(e * e, axis=1)[None, :]
    d2 = zsq - 2.0 * jnp.dot(zb, e.T, preferred_element_type=jnp.float32) + esq
    o_ref[...] = jnp.full((1, 1, 128), jnp.sum(jnp.min(d2, axis=1)),
                          jnp.float32)


def _shift3(xp):
    bb, hp, wp, c = xp.shape
    h = hp - 2
    return jnp.stack([xp[:, s:s + h, :, :].reshape(bb * h, wp, c)
                      for s in range(3)])


def _res_body(xs_ref, w3_ref, b3_ref, w1_ref, b1_ref, o_ref):
    m = _RH * _W
    acc = jnp.zeros((m, _D), jnp.float32)
    for dy in range(3):
        rows = xs_ref[dy]
        a = jnp.maximum(rows, 0.0)
        for dx in range(3):
            blk = a[:, dx:dx + _W, :].reshape(m, _D)
            acc = acc + jnp.dot(blk, w3_ref[dy, dx],
                                preferred_element_type=jnp.float32)
    y = jnp.maximum(acc + b3_ref[...], 0.0)
    y = jnp.dot(y, w1_ref[...], preferred_element_type=jnp.float32) + b1_ref[...]
    xc = xs_ref[1][:, 1:1 + _W, :].reshape(m, _D)
    o_ref[...] = xc + y


def _make_dc_body(ph, pw, w_cols, width):
    def body(xs_ref, wt_ref, b_ref, o_ref):
        m = _RH * width
        acc = jnp.zeros((m, w_cols), jnp.float32)
        for a in range(2):
            rows = xs_ref[ph + a]
            for b in range(2):
                blk = rows[:, pw + b:pw + b + width, :].reshape(m, _D)
                acc = acc + jnp.dot(blk, wt_ref[a, b],
                                    preferred_element_type=jnp.float32)
        o_ref[...] = acc + b_ref[...]
    return body


def _dc2_body(ys_ref, w_ref, b_ref, o_ref):
    m = _RH * _H2
    acc = jnp.zeros((m, 12), jnp.float32)
    for sh in range(3):
        rows = ys_ref[sh]
        for sw in range(3):
            blk = rows[:, sw:sw + _H2, :].reshape(m, _D)
            acc = acc + jnp.dot(blk, w_ref[sh, sw],
                                preferred_element_type=jnp.float32)
    o_ref[...] = acc + b_ref[...]


def kernel(z_e, embedding, r1_w3, r1_b3, r1_w1, r1_b1, r2_w3, r2_b3, r2_w1,
           r2_b1, dc1_w, dc1_b, dc2_w, dc2_b):
    f32 = jnp.float32
    x = jnp.transpose(z_e, (0, 2, 3, 1))
    z_flat = x.reshape(_N, _D)

    partials = pl.pallas_call(
        _vq_body,
        grid=(_N // _RB,),
        in_specs=[pl.BlockSpec((_RB, _D), lambda i: (i, 0)),
                  pl.BlockSpec((_K, _D), lambda i: (0, 0))],
        out_specs=pl.BlockSpec((1, 1, 128), lambda i: (i, 0, 0)),
        out_shape=jax.ShapeDtypeStruct((_N // _RB, 1, 128), f32),
    )(z_flat, embedding)
    vq_loss = 2.0 * jnp.sum(partials[:, 0, 0]) / (_N * _D)

    nrows = _B * _H
    grid_res = (nrows // _RH,)

    def resblock(xin, w3, b3, w1, b1):
        w3t = jnp.transpose(w3, (2, 3, 1, 0))
        w1t = jnp.transpose(w1[:, :, 0, 0], (1, 0))
        xs = _shift3(jnp.pad(xin, ((0, 0), (1, 1), (1, 1), (0, 0))))
        out = pl.pallas_call(
            _res_body,
            grid=grid_res,
            in_specs=[pl.BlockSpec((3, _RH, _W + 2, _D),
                                   lambda i: (0, i, 0, 0)),
                      pl.BlockSpec((3, 3, _D, _D), lambda i: (0, 0, 0, 0)),
                      pl.BlockSpec((1, _D), lambda i: (0, 0)),
                      pl.BlockSpec((_D, _D), lambda i: (0, 0)),
                      pl.BlockSpec((1, _D), lambda i: (0, 0))],
            out_specs=pl.BlockSpec((_RH * _W, _D), lambda i: (i, 0)),
            out_shape=jax.ShapeDtypeStruct((_N, _D), f32),
        )(xs, w3t, b3.reshape(1, _D), w1t, b1.reshape(1, _D))
        return out.reshape(_B, _H, _W, _D)

    x = resblock(x, r1_w3, r1_b3, r1_w1, r1_b1)
    x = resblock(x, r2_w3, r2_b3, r2_w1, r2_b1)

    wt = jnp.transpose(dc1_w, (2, 3, 1, 0))
    xs = _shift3(jnp.pad(x, ((0, 0), (1, 1), (1, 1), (0, 0))))
    pars = []
    for ph in (0, 1):
        for pw in (0, 1):
            tap = jnp.stack([jnp.stack([wt[2 * a + ph, 2 * b + pw]
                                        for b in (0, 1)]) for a in (0, 1)])
            pars.append(pl.pallas_call(
                _make_dc_body(ph, pw, _D, _W),
                grid=grid_res,
                in_specs=[pl.BlockSpec((3, _RH, _W + 2, _D),
                                       lambda i: (0, i, 0, 0)),
                          pl.BlockSpec((2, 2, _D, _D),
                                       lambda i: (0, 0, 0, 0)),
                          pl.BlockSpec((1, _D), lambda i: (0, 0))],
                out_specs=pl.BlockSpec((_RH * _W, _D), lambda i: (i, 0)),
                out_shape=jax.ShapeDtypeStruct((_N, _D), f32),
            )(xs, tap, dc1_b.reshape(1, _D)))
    par = jnp.stack(pars)
    y = par.reshape(2, 2, _B, _H, _W, _D).transpose(2, 3, 0, 4, 1, 5)
    y = y.reshape(_B, _H2, _H2, _D)

    w2t = jnp.transpose(dc2_w, (2, 3, 1, 0))
    w12 = jnp.zeros((3, 3, _D, 12), f32)
    for ph in (0, 1):
        for pw in (0, 1):
            for a in (0, 1):
                for b in (0, 1):
                    j = 6 * ph + 3 * pw
                    w12 = w12.at[a + ph, b + pw, :, j:j + 3].set(
                        w2t[2 * a + ph, 2 * b + pw])
    b12 = jnp.tile(dc2_b, 4).reshape(1, 12)
    ys = _shift3(jnp.pad(y, ((0, 0), (1, 1), (1, 1), (0, 0))))
    out12 = pl.pallas_call(
        _dc2_body,
        grid=(_B * _H2 // _RH,),
        in_specs=[pl.BlockSpec((3, _RH, _H2 + 2, _D),
                               lambda i: (0, i, 0, 0)),
                  pl.BlockSpec((3, 3, _D, 12), lambda i: (0, 0, 0, 0)),
                  pl.BlockSpec((1, 12), lambda i: (0, 0))],
        out_specs=pl.BlockSpec((_RH * _H2, 12), lambda i: (i, 0)),
        out_shape=jax.ShapeDtypeStruct((_N2, 12), f32),
    )(ys, w12, b12)
    xdec = out12.reshape(_B, _H2, _H2, 2, 2, 3).transpose(0, 1, 3, 2, 4, 5)
    xdec = xdec.reshape(_B, 2 * _H2, 2 * _H2, 3).transpose(0, 3, 1, 2)

    return (xdec, z_e, z_e, vq_loss)


# speedup vs baseline: 2.0699x; 2.0699x over previous
import functools

import jax
import jax.numpy as jnp
import numpy as np
from jax.experimental import pallas as pl

_einsum = functools.partial(jnp.einsum, precision=jax.lax.Precision.HIGHEST)

_B, _D, _H, _W = 2, 256, 56, 56
_K = 1024
_N = _B * _H * _W
_RB = 896
_RH = 8
_M = _RH * _W


def _vq_body(z_ref, e_ref, o_ref):
    zb = z_ref[...]
    e = e_ref[...]
    zsq = jnp.sum(zb * zb, axis=1, keepdims=True)
    esq = jnp.sum(e * e, axis=1)[None, :]
    d2 = zsq - 2.0 * jnp.dot(zb, e.T, preferred_element_type=jnp.float32) + esq
    o_ref[...] = jnp.full((1, 1, 128), jnp.sum(jnp.min(d2, axis=1)),
                          jnp.float32)


def _res_body(lo_ref, hi_ref, w3_ref, b3_ref, w1_ref, b1_ref, o_ref):
    chunk = jnp.concatenate([lo_ref[0], hi_ref[0]], axis=0)
    a = jnp.maximum(chunk, 0.0)
    acc = jnp.zeros((_M, _D), jnp.float32)
    for dy in range(3):
        for dx in range(3):
            blk = a[dy:dy + _RH, dx:dx + _W, :].reshape(_M, _D)
            acc = acc + jnp.dot(blk, w3_ref[dy, dx],
                                preferred_element_type=jnp.float32)
    y = jnp.maximum(acc + b3_ref[...], 0.0)
    y = jnp.dot(y, w1_ref[...], preferred_element_type=jnp.float32) + b1_ref[...]
    xc = chunk[1:1 + _RH, 1:1 + _W, :].reshape(_M, _D)
    o_ref[0] = (xc + y).reshape(_RH, _W, _D)


def _tail_body(lo_ref, hi_ref, w_ref, o_ref):
    chunk = jnp.concatenate([lo_ref[0], hi_ref[0]], axis=0)
    acc = jnp.zeros((_M, 48), jnp.float32)
    for sh in range(3):
        for sw in range(3):
            blk = chunk[sh:sh + _RH, sw:sw + _W, :].reshape(_M, _D)
            acc = acc + jnp.dot(blk, w_ref[sh, sw],
                                preferred_element_type=jnp.float32)
    o_ref[0] = acc.reshape(_RH, _W, 48)


def _halo_call(body, xin, consts, const_specs, out_ch):
    xp = jnp.pad(xin, ((0, 0), (1, 7), (1, 1), (0, 0)))
    blk = (1, _RH, _W + 2, _D)
    return pl.pallas_call(
        body,
        grid=(_B, _H // _RH),
        in_specs=[pl.BlockSpec(blk, lambda b, j: (b, j, 0, 0)),
                  pl.BlockSpec(blk, lambda b, j: (b, j + 1, 0, 0))]
        + const_specs,
        out_specs=pl.BlockSpec((1, _RH, _W, out_ch),
                               lambda b, j: (b, j, 0, 0)),
        out_shape=jax.ShapeDtypeStruct((_B, _H, _W, out_ch), jnp.float32),
    )(xp, xp, *consts)


def kernel(z_e, embedding, r1_w3, r1_b3, r1_w1, r1_b1, r2_w3, r2_b3, r2_w1,
           r2_b1, dc1_w, dc1_b, dc2_w, dc2_b):
    f32 = jnp.float32
    x = jnp.transpose(z_e, (0, 2, 3, 1))
    z_flat = x.reshape(_N, _D)

    partials = pl.pallas_call(
        _vq_body,
        grid=(_N // _RB,),
        in_specs=[pl.BlockSpec((_RB, _D), lambda i: (i, 0)),
                  pl.BlockSpec((_K, _D), lambda i: (0, 0))],
        out_specs=pl.BlockSpec((1, 1, 128), lambda i: (i, 0, 0)),
        out_shape=jax.ShapeDtypeStruct((_N // _RB, 1, 128), f32),
    )(z_flat, embedding)
    vq_loss = 2.0 * jnp.sum(partials[:, 0, 0]) / (_N * _D)

    wspec = pl.BlockSpec((3, 3, _D, _D), lambda b, j: (0, 0, 0, 0))
    bspec = pl.BlockSpec((1, _D), lambda b, j: (0, 0))
    w1spec = pl.BlockSpec((_D, _D), lambda b, j: (0, 0))

    def resblock(xin, w3, b3, w1, b1):
        w3t = jnp.transpose(w3, (2, 3, 1, 0))
        w1t = jnp.transpose(w1[:, :, 0, 0], (1, 0))
        return _halo_call(
            _res_body, xin,
            [w3t, b3.reshape(1, _D), w1t, b1.reshape(1, _D)],
            [wspec, bspec, w1spec, bspec], _D)

    x = resblock(x, r1_w3, r1_b3, r1_w1, r1_b1)
    x = resblock(x, r2_w3, r2_b3, r2_w1, r2_b1)

    wp = _einsum('jiab,cjde->adbeic', dc1_w, dc2_w)
    wp16 = wp.reshape(16, 16, _D, 3)
    mt = np.zeros((16, 10), np.float32)
    for k1 in range(4):
        for k2 in range(4):
            mt[4 * k1 + k2, 9 - 2 * k1 - k2] = 1.0
    mt = jnp.asarray(mt)
    weff = _einsum('pqic,pt,qu->tuic', wp16, mt, mt)

    cols = []
    for sh in range(3):
        row = []
        for sw in range(3):
            phase = []
            for rh in range(4):
                for rw in range(4):
                    th = rh - 4 * sh + 4
                    tw = rw - 4 * sw + 4
                    if -3 <= th <= 6 and -3 <= tw <= 6:
                        phase.append(weff[th + 3, tw + 3])
                    else:
                        phase.append(jnp.zeros((_D, 3), f32))
            row.append(jnp.concatenate(phase, axis=1))
        cols.append(jnp.stack(row))
    w48 = jnp.stack(cols)

    w48spec = pl.BlockSpec((3, 3, _D, 48), lambda b, j: (0, 0, 0, 0))
    out48 = _halo_call(_tail_body, x, [w48], [w48spec], 48)

    sbw = _einsum('j,cjkl->klc', dc1_b, dc2_w)
    ih = np.zeros((224, 4), np.float32)
    for o in range(224):
        for k2 in range(4):
            if (o + k2) % 2 == 0 and 0 <= (o + k2 - 2) // 2 <= 111:
                ih[o, k2] = 1.0
    ih = jnp.asarray(ih)
    bfield = _einsum('ok,klc,pl->opc', ih, sbw, ih)

    xdec = out48.reshape(_B, _H, _W, 4, 4, 3).transpose(0, 1, 3, 2, 4, 5)
    xdec = xdec.reshape(_B, 224, 224, 3)
    xdec = xdec + bfield[None] + dc2_b[None, None, None, :]

    def edge_eff(pair44):
        return _einsum('pic,pt->tic', pair44.reshape(16, _D, 3), mt)

    def edge_apply(xline, veff):
        xlp = jnp.pad(xline, ((0, 0), (1, 1), (0, 0)))
        acc = jnp.zeros((_B, _W, 12), f32)
        for s in range(3):
            wc = []
            for r in range(4):
                t = r - 4 * s + 4
                wc.append(veff[t + 3] if -3 <= t <= 6
                          else jnp.zeros((_D, 3), f32))
            wmat = jnp.concatenate(wc, axis=1)
            acc = acc + _einsum('bwi,ic->bwc', xlp[:, s:s + _W, :], wmat)
        return acc.reshape(_B, 224, 3)

    v_top = edge_eff(_einsum('jib,cjd->bdic', dc1_w[:, :, 3, :],
                                dc2_w[:, :, 0, :]))
    v_bot = edge_eff(_einsum('jib,cjd->bdic', dc1_w[:, :, 0, :],
                                dc2_w[:, :, 3, :]))
    v_lft = edge_eff(_einsum('jia,cjd->adic', dc1_w[:, :, :, 3],
                                dc2_w[:, :, :, 0]))
    v_rgt = edge_eff(_einsum('jia,cjd->adic', dc1_w[:, :, :, 0],
                                dc2_w[:, :, :, 3]))
    a_top = edge_apply(x[:, 0, :, :], v_top)
    a_bot = edge_apply(x[:, 55, :, :], v_bot)
    a_lft = edge_apply(x[:, :, 0, :], v_lft)
    a_rgt = edge_apply(x[:, :, 55, :], v_rgt)

    def corner(px, k1h, k1w, k2h, k2w):
        m = _einsum('ji,cj->ic', dc1_w[:, :, k1h, k1w],
                       dc2_w[:, :, k2h, k2w])
        return _einsum('bi,ic->bc', px, m)

    c00 = corner(x[:, 0, 0, :], 3, 3, 0, 0)
    c0r = corner(x[:, 0, 55, :], 3, 0, 0, 3)
    cr0 = corner(x[:, 55, 0, :], 0, 3, 3, 0)
    crr = corner(x[:, 55, 55, :], 0, 0, 3, 3)

    xdec = xdec.at[:, 0, :, :].add(-a_top)
    xdec = xdec.at[:, 223, :, :].add(-a_bot)
    xdec = xdec.at[:, :, 0, :].add(-a_lft)
    xdec = xdec.at[:, :, 223, :].add(-a_rgt)
    xdec = xdec.at[:, 0, 0, :].add(c00)
    xdec = xdec.at[:, 0, 223, :].add(c0r)
    xdec = xdec.at[:, 223, 0, :].add(cr0)
    xdec = xdec.at[:, 223, 223, :].add(crr)
    xdec = xdec.transpose(0, 3, 1, 2)

    return (xdec, z_e, z_e, vq_loss)


# speedup vs baseline: 2.3073x; 1.1147x over previous
import functools

import jax
import jax.numpy as jnp
import numpy as np
from jax.experimental import pallas as pl

_einsum = functools.partial(jnp.einsum, precision=jax.lax.Precision.HIGHEST)

_B, _D, _H, _W = 2, 256, 56, 56
_K = 1024
_N = _B * _H * _W
_RB = 896
_RH = 8
_M = _RH * _W


def _vq_body(z_ref, e_ref, o_ref):
    zb = z_ref[...]
    e = e_ref[...]
    zsq = jnp.sum(zb * zb, axis=1, keepdims=True)
    esq = jnp.sum(e * e, axis=1)[None, :]
    d2 = zsq - 2.0 * jnp.dot(zb, e.T, preferred_element_type=jnp.float32) + esq
    o_ref[...] = jnp.full((1, 1, 128), jnp.sum(jnp.min(d2, axis=1)),
                          jnp.float32)


def _res_core(lo_ref, hi_ref, w3_ref, b3_ref, w1_ref, b1_ref):
    chunk = jnp.concatenate([lo_ref[0], hi_ref[0]], axis=0)
    a = jnp.maximum(chunk, 0.0)
    acc = jnp.zeros((_M, _D), jnp.float32)
    for dy in range(3):
        for dx in range(3):
            blk = a[dy:dy + _RH, dx:dx + _W, :].reshape(_M, _D)
            acc = acc + jnp.dot(blk, w3_ref[dy, dx],
                                preferred_element_type=jnp.float32)
    y = jnp.maximum(acc + b3_ref[...], 0.0)
    y = jnp.dot(y, w1_ref[...], preferred_element_type=jnp.float32) + b1_ref[...]
    xc = chunk[1:1 + _RH, 1:1 + _W, :].reshape(_M, _D)
    return xc, y


def _res_body(lo_ref, hi_ref, w3_ref, b3_ref, w1_ref, b1_ref, o_ref):
    xc, y = _res_core(lo_ref, hi_ref, w3_ref, b3_ref, w1_ref, b1_ref)
    o_ref[0] = (xc + y).reshape(_RH, _W, _D)


def _res_vq_body(lo_ref, hi_ref, e_ref, w3_ref, b3_ref, w1_ref, b1_ref,
                 o_ref, vq_ref):
    xc, y = _res_core(lo_ref, hi_ref, w3_ref, b3_ref, w1_ref, b1_ref)
    o_ref[0] = (xc + y).reshape(_RH, _W, _D)
    e = e_ref[...]
    zsq = jnp.sum(xc * xc, axis=1, keepdims=True)
    esq = jnp.sum(e * e, axis=1)[None, :]
    d2 = zsq - 2.0 * jnp.dot(xc, e.T, preferred_element_type=jnp.float32) + esq
    vq_ref[...] = jnp.full((1, 1, 1, 128), jnp.sum(jnp.min(d2, axis=1)),
                           jnp.float32)


def _tail_body(lo_ref, hi_ref, w_ref, o_ref):
    chunk = jnp.concatenate([lo_ref[0], hi_ref[0]], axis=0)
    acc = jnp.zeros((_M, 48), jnp.float32)
    for sh in range(3):
        for sw in range(3):
            blk = chunk[sh:sh + _RH, sw:sw + _W, :].reshape(_M, _D)
            acc = acc + jnp.dot(blk, w_ref[sh, sw],
                                preferred_element_type=jnp.float32)
    o_ref[0] = acc.reshape(_RH, _W, 48)


def _halo_call(body, xin, consts, const_specs, out_ch):
    xp = jnp.pad(xin, ((0, 0), (1, 7), (1, 1), (0, 0)))
    blk = (1, _RH, _W + 2, _D)
    return pl.pallas_call(
        body,
        grid=(_B, _H // _RH),
        in_specs=[pl.BlockSpec(blk, lambda b, j: (b, j, 0, 0)),
                  pl.BlockSpec(blk, lambda b, j: (b, j + 1, 0, 0))]
        + const_specs,
        out_specs=pl.BlockSpec((1, _RH, _W, out_ch),
                               lambda b, j: (b, j, 0, 0)),
        out_shape=jax.ShapeDtypeStruct((_B, _H, _W, out_ch), jnp.float32),
    )(xp, xp, *consts)


def kernel(z_e, embedding, r1_w3, r1_b3, r1_w1, r1_b1, r2_w3, r2_b3, r2_w1,
           r2_b1, dc1_w, dc1_b, dc2_w, dc2_b):
    f32 = jnp.float32
    x = jnp.transpose(z_e, (0, 2, 3, 1))

    wspec = pl.BlockSpec((3, 3, _D, _D), lambda b, j: (0, 0, 0, 0))
    bspec = pl.BlockSpec((1, _D), lambda b, j: (0, 0))
    w1spec = pl.BlockSpec((_D, _D), lambda b, j: (0, 0))

    def res_weights(w3, b3, w1, b1):
        w3t = jnp.transpose(w3, (2, 3, 1, 0))
        w1t = jnp.transpose(w1[:, :, 0, 0], (1, 0))
        return [w3t, b3.reshape(1, _D), w1t, b1.reshape(1, _D)]

    xp = jnp.pad(x, ((0, 0), (1, 7), (1, 1), (0, 0)))
    blk = (1, _RH, _W + 2, _D)
    x, partials = pl.pallas_call(
        _res_vq_body,
        grid=(_B, _H // _RH),
        in_specs=[pl.BlockSpec(blk, lambda b, j: (b, j, 0, 0)),
                  pl.BlockSpec(blk, lambda b, j: (b, j + 1, 0, 0)),
                  pl.BlockSpec((_K, _D), lambda b, j: (0, 0)),
                  wspec, bspec, w1spec, bspec],
        out_specs=[pl.BlockSpec((1, _RH, _W, _D), lambda b, j: (b, j, 0, 0)),
                   pl.BlockSpec((1, 1, 1, 128), lambda b, j: (b, j, 0, 0))],
        out_shape=[jax.ShapeDtypeStruct((_B, _H, _W, _D), f32),
                   jax.ShapeDtypeStruct((_B, _H // _RH, 1, 128), f32)],
    )(xp, xp, embedding, *res_weights(r1_w3, r1_b3, r1_w1, r1_b1))
    vq_loss = 2.0 * jnp.sum(partials[:, :, 0, 0]) / (_N * _D)

    x = _halo_call(_res_body, x, res_weights(r2_w3, r2_b3, r2_w1, r2_b1),
                   [wspec, bspec, w1spec, bspec], _D)

    wp = _einsum('jiab,cjde->adbeic', dc1_w, dc2_w)
    wp16 = wp.reshape(16, 16, _D, 3)
    mt = np.zeros((16, 10), np.float32)
    for k1 in range(4):
        for k2 in range(4):
            mt[4 * k1 + k2, 9 - 2 * k1 - k2] = 1.0
    mt = jnp.asarray(mt)
    weff = _einsum('pqic,pt,qu->tuic', wp16, mt, mt)

    cols = []
    for sh in range(3):
        row = []
        for sw in range(3):
            phase = []
            for rh in range(4):
                for rw in range(4):
                    th = rh - 4 * sh + 4
                    tw = rw - 4 * sw + 4
                    if -3 <= th <= 6 and -3 <= tw <= 6:
                        phase.append(weff[th + 3, tw + 3])
                    else:
                        phase.append(jnp.zeros((_D, 3), f32))
            row.append(jnp.concatenate(phase, axis=1))
        cols.append(jnp.stack(row))
    w48 = jnp.stack(cols)

    w48spec = pl.BlockSpec((3, 3, _D, 48), lambda b, j: (0, 0, 0, 0))
    out48 = _halo_call(_tail_body, x, [w48], [w48spec], 48)

    sbw = _einsum('j,cjkl->klc', dc1_b, dc2_w)
    ih = np.zeros((224, 4), np.float32)
    for o in range(224):
        for k2 in range(4):
            if (o + k2) % 2 == 0 and 0 <= (o + k2 - 2) // 2 <= 111:
                ih[o, k2] = 1.0
    ih = jnp.asarray(ih)
    bfield = _einsum('ok,klc,pl->opc', ih, sbw, ih)

    xdec = out48.reshape(_B, _H, _W, 4, 4, 3).transpose(0, 1, 3, 2, 4, 5)
    xdec = xdec.reshape(_B, 224, 224, 3)
    xdec = xdec + bfield[None] + dc2_b[None, None, None, :]

    def edge_eff(pair44):
        return _einsum('pic,pt->tic', pair44.reshape(16, _D, 3), mt)

    def edge_apply(xline, veff):
        xlp = jnp.pad(xline, ((0, 0), (1, 1), (0, 0)))
        acc = jnp.zeros((_B, _W, 12), f32)
        for s in range(3):
            wc = []
            for r in range(4):
                t = r - 4 * s + 4
                wc.append(veff[t + 3] if -3 <= t <= 6
                          else jnp.zeros((_D, 3), f32))
            wmat = jnp.concatenate(wc, axis=1)
            acc = acc + _einsum('bwi,ic->bwc', xlp[:, s:s + _W, :], wmat)
        return acc.reshape(_B, 224, 3)

    v_top = edge_eff(_einsum('jib,cjd->bdic', dc1_w[:, :, 3, :],
                                dc2_w[:, :, 0, :]))
    v_bot = edge_eff(_einsum('jib,cjd->bdic', dc1_w[:, :, 0, :],
                                dc2_w[:, :, 3, :]))
    v_lft = edge_eff(_einsum('jia,cjd->adic', dc1_w[:, :, :, 3],
                                dc2_w[:, :, :, 0]))
    v_rgt = edge_eff(_einsum('jia,cjd->adic', dc1_w[:, :, :, 0],
                                dc2_w[:, :, :, 3]))
    a_top = edge_apply(x[:, 0, :, :], v_top)
    a_bot = edge_apply(x[:, 55, :, :], v_bot)
    a_lft = edge_apply(x[:, :, 0, :], v_lft)
    a_rgt = edge_apply(x[:, :, 55, :], v_rgt)

    def corner(px, k1h, k1w, k2h, k2w):
        m = _einsum('ji,cj->ic', dc1_w[:, :, k1h, k1w],
                       dc2_w[:, :, k2h, k2w])
        return _einsum('bi,ic->bc', px, m)

    c00 = corner(x[:, 0, 0, :], 3, 3, 0, 0)
    c0r = corner(x[:, 0, 55, :], 3, 0, 0, 3)
    cr0 = corner(x[:, 55, 0, :], 0, 3, 3, 0)
    crr = corner(x[:, 55, 55, :], 0, 0, 3, 3)

    xdec = xdec.at[:, 0, :, :].add(-a_top)
    xdec = xdec.at[:, 223, :, :].add(-a_bot)
    xdec = xdec.at[:, :, 0, :].add(-a_lft)
    xdec = xdec.at[:, :, 223, :].add(-a_rgt)
    xdec = xdec.at[:, 0, 0, :].add(c00)
    xdec = xdec.at[:, 0, 223, :].add(c0r)
    xdec = xdec.at[:, 223, 0, :].add(cr0)
    xdec = xdec.at[:, 223, 223, :].add(crr)
    xdec = xdec.transpose(0, 3, 1, 2)

    return (xdec, z_e, z_e, vq_loss)


# speedup vs baseline: 2.4189x; 1.0484x over previous
import functools

import jax
import jax.numpy as jnp
import numpy as np
from jax.experimental import pallas as pl

_einsum = functools.partial(jnp.einsum, precision=jax.lax.Precision.HIGHEST)

_B, _D, _H, _W = 2, 256, 56, 56
_K = 1024
_N = _B * _H * _W
_RB = 896
_RH = 8
_M = _B * _RH * _W


def _vq_body(z_ref, e_ref, o_ref):
    zb = z_ref[...]
    e = e_ref[...]
    zsq = jnp.sum(zb * zb, axis=1, keepdims=True)
    esq = jnp.sum(e * e, axis=1)[None, :]
    d2 = zsq - 2.0 * jnp.dot(zb, e.T, preferred_element_type=jnp.float32) + esq
    o_ref[...] = jnp.full((1, 1, 128), jnp.sum(jnp.min(d2, axis=1)),
                          jnp.float32)


def _res_core(lo_ref, hi_ref, w3_ref, b3_ref, w1_ref, b1_ref):
    chunk = jnp.concatenate([lo_ref[...], hi_ref[...]], axis=1)
    a = jnp.maximum(chunk, 0.0)
    acc = jnp.zeros((_M, _D), jnp.float32)
    for dy in range(3):
        for dx in range(3):
            blk = a[:, dy:dy + _RH, dx:dx + _W, :].reshape(_M, _D)
            acc = acc + jnp.dot(blk, w3_ref[dy, dx],
                                preferred_element_type=jnp.float32)
    y = jnp.maximum(acc + b3_ref[...], 0.0)
    y = jnp.dot(y, w1_ref[...], preferred_element_type=jnp.float32) + b1_ref[...]
    xc = chunk[:, 1:1 + _RH, 1:1 + _W, :].reshape(_M, _D)
    return xc, y


def _res_body(lo_ref, hi_ref, w3_ref, b3_ref, w1_ref, b1_ref, o_ref):
    xc, y = _res_core(lo_ref, hi_ref, w3_ref, b3_ref, w1_ref, b1_ref)
    o_ref[...] = (xc + y).reshape(_B, _RH, _W, _D)


def _res_vq_body(lo_ref, hi_ref, e_ref, w3_ref, b3_ref, w1_ref, b1_ref,
                 o_ref, vq_ref):
    xc, y = _res_core(lo_ref, hi_ref, w3_ref, b3_ref, w1_ref, b1_ref)
    o_ref[...] = (xc + y).reshape(_B, _RH, _W, _D)
    e = e_ref[...]
    zsq = jnp.sum(xc * xc, axis=1, keepdims=True)
    esq = jnp.sum(e * e, axis=1)[None, :]
    d2 = zsq - 2.0 * jnp.dot(xc, e.T, preferred_element_type=jnp.float32) + esq
    vq_ref[...] = jnp.full((1, 1, 128), jnp.sum(jnp.min(d2, axis=1)),
                           jnp.float32)


def _tail_body(lo_ref, hi_ref, w_ref, o_ref):
    chunk = jnp.concatenate([lo_ref[...], hi_ref[...]], axis=1)
    acc = jnp.zeros((_M, 48), jnp.float32)
    for sh in range(3):
        for sw in range(3):
            blk = chunk[:, sh:sh + _RH, sw:sw + _W, :].reshape(_M, _D)
            acc = acc + jnp.dot(blk, w_ref[sh, sw],
                                preferred_element_type=jnp.float32)
    o_ref[...] = acc.reshape(_B, _RH, _W, 48)


def _halo_call(body, xin, consts, const_specs, out_ch):
    xp = jnp.pad(xin, ((0, 0), (1, 7), (1, 1), (0, 0)))
    blk = (_B, _RH, _W + 2, _D)
    return pl.pallas_call(
        body,
        grid=(_H // _RH,),
        in_specs=[pl.BlockSpec(blk, lambda j: (0, j, 0, 0)),
                  pl.BlockSpec(blk, lambda j: (0, j + 1, 0, 0))]
        + const_specs,
        out_specs=pl.BlockSpec((_B, _RH, _W, out_ch),
                               lambda j: (0, j, 0, 0)),
        out_shape=jax.ShapeDtypeStruct((_B, _H, _W, out_ch), jnp.float32),
    )(xp, xp, *consts)


def kernel(z_e, embedding, r1_w3, r1_b3, r1_w1, r1_b1, r2_w3, r2_b3, r2_w1,
           r2_b1, dc1_w, dc1_b, dc2_w, dc2_b):
    f32 = jnp.float32
    x = jnp.transpose(z_e, (0, 2, 3, 1))

    wspec = pl.BlockSpec((3, 3, _D, _D), lambda j: (0, 0, 0, 0))
    bspec = pl.BlockSpec((1, _D), lambda j: (0, 0))
    w1spec = pl.BlockSpec((_D, _D), lambda j: (0, 0))

    def res_weights(w3, b3, w1, b1):
        w3t = jnp.transpose(w3, (2, 3, 1, 0))
        w1t = jnp.transpose(w1[:, :, 0, 0], (1, 0))
        return [w3t, b3.reshape(1, _D), w1t, b1.reshape(1, _D)]

    xp = jnp.pad(x, ((0, 0), (1, 7), (1, 1), (0, 0)))
    blk = (_B, _RH, _W + 2, _D)
    x, partials = pl.pallas_call(
        _res_vq_body,
        grid=(_H // _RH,),
        in_specs=[pl.BlockSpec(blk, lambda j: (0, j, 0, 0)),
                  pl.BlockSpec(blk, lambda j: (0, j + 1, 0, 0)),
                  pl.BlockSpec((_K, _D), lambda j: (0, 0)),
                  wspec, bspec, w1spec, bspec],
        out_specs=[pl.BlockSpec((_B, _RH, _W, _D), lambda j: (0, j, 0, 0)),
                   pl.BlockSpec((1, 1, 128), lambda j: (j, 0, 0))],
        out_shape=[jax.ShapeDtypeStruct((_B, _H, _W, _D), f32),
                   jax.ShapeDtypeStruct((_H // _RH, 1, 128), f32)],
    )(xp, xp, embedding, *res_weights(r1_w3, r1_b3, r1_w1, r1_b1))
    vq_loss = 2.0 * jnp.sum(partials[:, 0, 0]) / (_N * _D)

    x = _halo_call(_res_body, x, res_weights(r2_w3, r2_b3, r2_w1, r2_b1),
                   [wspec, bspec, w1spec, bspec], _D)

    wp = _einsum('jiab,cjde->adbeic', dc1_w, dc2_w)
    wp16 = wp.reshape(16, 16, _D, 3)
    mt = np.zeros((16, 10), np.float32)
    for k1 in range(4):
        for k2 in range(4):
            mt[4 * k1 + k2, 9 - 2 * k1 - k2] = 1.0
    mt = jnp.asarray(mt)
    weff = _einsum('pqic,pt,qu->tuic', wp16, mt, mt)

    cols = []
    for sh in range(3):
        row = []
        for sw in range(3):
            phase = []
            for rh in range(4):
                for rw in range(4):
                    th = rh - 4 * sh + 4
                    tw = rw - 4 * sw + 4
                    if -3 <= th <= 6 and -3 <= tw <= 6:
                        phase.append(weff[th + 3, tw + 3])
                    else:
                        phase.append(jnp.zeros((_D, 3), f32))
            row.append(jnp.concatenate(phase, axis=1))
        cols.append(jnp.stack(row))
    w48 = jnp.stack(cols)

    w48spec = pl.BlockSpec((3, 3, _D, 48), lambda j: (0, 0, 0, 0))
    out48 = _halo_call(_tail_body, x, [w48], [w48spec], 48)

    sbw = _einsum('j,cjkl->klc', dc1_b, dc2_w)
    ih = np.zeros((224, 4), np.float32)
    for o in range(224):
        for k2 in range(4):
            if (o + k2) % 2 == 0 and 0 <= (o + k2 - 2) // 2 <= 111:
                ih[o, k2] = 1.0
    ih = jnp.asarray(ih)
    bfield = _einsum('ok,klc,pl->opc', ih, sbw, ih)

    xdec = out48.reshape(_B, _H, _W, 4, 4, 3).transpose(0, 1, 3, 2, 4, 5)
    xdec = xdec.reshape(_B, 224, 224, 3)
    xdec = xdec + bfield[None] + dc2_b[None, None, None, :]

    def edge_eff(pair44):
        return _einsum('pic,pt->tic', pair44.reshape(16, _D, 3), mt)

    def edge_apply(xline, veff):
        xlp = jnp.pad(xline, ((0, 0), (1, 1), (0, 0)))
        acc = jnp.zeros((_B, _W, 12), f32)
        for s in range(3):
            wc = []
            for r in range(4):
                t = r - 4 * s + 4
                wc.append(veff[t + 3] if -3 <= t <= 6
                          else jnp.zeros((_D, 3), f32))
            wmat = jnp.concatenate(wc, axis=1)
            acc = acc + _einsum('bwi,ic->bwc', xlp[:, s:s + _W, :], wmat)
        return acc.reshape(_B, 224, 3)

    v_top = edge_eff(_einsum('jib,cjd->bdic', dc1_w[:, :, 3, :],
                                dc2_w[:, :, 0, :]))
    v_bot = edge_eff(_einsum('jib,cjd->bdic', dc1_w[:, :, 0, :],
                                dc2_w[:, :, 3, :]))
    v_lft = edge_eff(_einsum('jia,cjd->adic', dc1_w[:, :, :, 3],
                                dc2_w[:, :, :, 0]))
    v_rgt = edge_eff(_einsum('jia,cjd->adic', dc1_w[:, :, :, 0],
                                dc2_w[:, :, :, 3]))
    a_top = edge_apply(x[:, 0, :, :], v_top)
    a_bot = edge_apply(x[:, 55, :, :], v_bot)
    a_lft = edge_apply(x[:, :, 0, :], v_lft)
    a_rgt = edge_apply(x[:, :, 55, :], v_rgt)

    def corner(px, k1h, k1w, k2h, k2w):
        m = _einsum('ji,cj->ic', dc1_w[:, :, k1h, k1w],
                       dc2_w[:, :, k2h, k2w])
        return _einsum('bi,ic->bc', px, m)

    c00 = corner(x[:, 0, 0, :], 3, 3, 0, 0)
    c0r = corner(x[:, 0, 55, :], 3, 0, 0, 3)
    cr0 = corner(x[:, 55, 0, :], 0, 3, 3, 0)
    crr = corner(x[:, 55, 55, :], 0, 0, 3, 3)

    xdec = xdec.at[:, 0, :, :].add(-a_top)
    xdec = xdec.at[:, 223, :, :].add(-a_bot)
    xdec = xdec.at[:, :, 0, :].add(-a_lft)
    xdec = xdec.at[:, :, 223, :].add(-a_rgt)
    xdec = xdec.at[:, 0, 0, :].add(c00)
    xdec = xdec.at[:, 0, 223, :].add(c0r)
    xdec = xdec.at[:, 223, 0, :].add(cr0)
    xdec = xdec.at[:, 223, 223, :].add(crr)
    xdec = xdec.transpose(0, 3, 1, 2)

    return (xdec, z_e, z_e, vq_loss)
